# baseline (speedup 1.0000x reference)
import jax
import jax.numpy as jnp
from jax import lax
from jax.experimental import pallas as pl
from jax.experimental.pallas import tpu as pltpu
from jax.experimental.pallas import tpu_sc as plsc

_NC = 2
_NS = 16
_NW = _NC * _NS
_C = 125
_C3 = 625
_NB = 16


def _dot_t(a, w):
    return lax.dot_general(a, w, (((1,), (1,)), ((), ())),
                           preferred_element_type=jnp.float32)


def _tc_mm(a, w):
    n = a.shape[0]
    do = w.shape[0]

    def body(a_ref, w_ref, y_ref):
        y_ref[...] = _dot_t(a_ref[...], w_ref[...])

    return pl.pallas_call(
        body,
        out_shape=jax.ShapeDtypeStruct((n, do), jnp.float32),
    )(a, w)


def _tc_mid(sp, cp, z, b, g, be, wl):
    n = sp.shape[1]
    do = wl.shape[0]
    dh = sp.shape[2]

    def body(sp_ref, cp_ref, z_ref, b_ref, g_ref, be_ref, wl_ref,
             y_ref, h_ref):
        cnt = cp_ref[0] + cp_ref[1]
        inv = 1.0 / jnp.maximum(cnt[:, 0:1], 1.0)
        s = sp_ref[0] + sp_ref[1]
        u = s * inv + b_ref[...][None, :] + z_ref[...]
        m = jnp.mean(u, axis=0, keepdims=True)
        v = jnp.mean((u - m) ** 2, axis=0, keepdims=True)
        h = (u - m) * lax.rsqrt(v + 1e-5) * g_ref[...][None, :] + be_ref[...][None, :]
        h = jnp.maximum(h, 0.0)
        h_ref[...] = h
        y_ref[...] = _dot_t(h, wl_ref[...])

    return pl.pallas_call(
        body,
        out_shape=(jax.ShapeDtypeStruct((n, do), jnp.float32),
                   jax.ShapeDtypeStruct((n, dh), jnp.float32)),
    )(sp, cp, z, b, g, be, wl)


def _tc_post(sp, cp, z, b):
    n = sp.shape[1]
    do = sp.shape[2]

    def body(sp_ref, cp_ref, z_ref, b_ref, o_ref):
        cnt = cp_ref[0] + cp_ref[1]
        inv = 1.0 / jnp.maximum(cnt[:, 0:1], 1.0)
        u = (sp_ref[0] + sp_ref[1]) * inv + b_ref[...][None, :] + z_ref[...]
        mx = jnp.max(u, axis=1, keepdims=True)
        lse = mx + jnp.log(jnp.sum(jnp.exp(u - mx), axis=1, keepdims=True))
        o_ref[...] = u - lse

    return pl.pallas_call(
        body,
        out_shape=jax.ShapeDtypeStruct((n, do), jnp.float32),
    )(sp, cp, z, b)


def _sliced_copy(sid, rps, rem, src_ref, dst_ref):
    zb = sid * rps
    pltpu.sync_copy(src_ref.at[pl.ds(zb, rps)], dst_ref.at[pl.ds(zb, rps)])
    if rem:
        @pl.when(sid == 0)
        def _():
            pltpu.sync_copy(src_ref.at[pl.ds(_NS * rps, rem)],
                            dst_ref.at[pl.ds(_NS * rps, rem)])


def _sc_count(dst_r, zeros16, ones16):
    n = zeros16.shape[0]
    nch, c = dst_r.shape[1], dst_r.shape[2]
    nb = min(_NB, nch)
    nblk = nch // nb
    rps = (n // _NS) & ~7
    rem = n - _NS * rps
    mesh = plsc.VectorSubcoreMesh(core_axis_name="c", subcore_axis_name="s",
                                  num_cores=_NC, num_subcores=_NS)
    scratch = [
        pltpu.VMEM((nb, c), jnp.int32),
        pltpu.VMEM((c, 16), jnp.float32),
        pltpu.VMEM_SHARED((n, 16), jnp.float32),
        pltpu.SemaphoreType.DMA,
    ]

    def body(dst_hbm, z16_hbm, ones_hbm, cnt_hbm, dst_v, ones_v, cacc, sem):
        cid = lax.axis_index("c")
        sid = lax.axis_index("s")
        wid = cid * _NS + sid
        _sliced_copy(sid, rps, rem, z16_hbm, cacc)
        pltpu.sync_copy(ones_hbm, ones_v)
        plsc.subcore_barrier()

        def step_blk(bi, carry):
            pltpu.sync_copy(dst_hbm.at[wid, pl.ds(bi * nb, nb)], dst_v)
            descs = [pltpu.async_copy(ones_v, cacc.at[dst_v.at[j]], sem,
                                      add=True)
                     for j in range(nb)]
            for dsc in descs:
                dsc.wait()
            return carry

        lax.fori_loop(0, nblk, step_blk, 0)
        plsc.subcore_barrier()
        _sliced_copy(sid, rps, rem, cacc, cnt_hbm.at[cid])

    return pl.kernel(body,
                     out_type=(jax.ShapeDtypeStruct((_NC, n, 16),
                                                    jnp.float32),),
                     mesh=mesh, scratch_types=scratch,
                     compiler_params=pltpu.CompilerParams(
                         use_tc_tiling_on_sc=False))(dst_r, zeros16, ones16)[0]


def _sc_agg(y, src_r, dst_r, zeros_d):
    n, d = y.shape
    nch, c = src_r.shape[1], src_r.shape[2]
    nb = min(_NB, nch)
    nblk = nch // nb
    rps = (n // _NS) & ~7
    rem = n - _NS * rps

    mesh = plsc.VectorSubcoreMesh(core_axis_name="c", subcore_axis_name="s",
                                  num_cores=_NC, num_subcores=_NS)
    nbuf = 3 if d <= 16 else 2
    scratch = [
        pltpu.VMEM((nb, c), jnp.int32),
        pltpu.VMEM((nb, c), jnp.int32),
        pltpu.VMEM((nbuf, c, d), jnp.float32),
        pltpu.VMEM_SHARED((n, d), jnp.float32),
    ] + [pltpu.SemaphoreType.DMA] * (2 * nbuf)

    def body(y_hbm, src_hbm, dst_hbm, zd_hbm, out_hbm,
             src_v, dst_v, rows_v, acc, *sems):
        cid = lax.axis_index("c")
        sid = lax.axis_index("s")
        wid = cid * _NS + sid
        gsems = sems[:nbuf]
        ssems = sems[nbuf:]
        _sliced_copy(sid, rps, rem, zd_hbm, acc)
        plsc.subcore_barrier()

        def step_blk(bi, carry):
            pltpu.sync_copy(src_hbm.at[wid, pl.ds(bi * nb, nb)], src_v)
            pltpu.sync_copy(dst_hbm.at[wid, pl.ds(bi * nb, nb)], dst_v)
            gd = [None] * nb
            sd = [None] * nb
            for j in range(min(nbuf - 1, nb)):
                gd[j] = pltpu.async_copy(y_hbm.at[src_v.at[j]],
                                         rows_v.at[j % nbuf], gsems[j % nbuf])
            for j in range(nb):
                p = j % nbuf
                jn = j + nbuf - 1
                if jn < nb:
                    q = jn % nbuf
                    if j >= 1:
                        sd[j - 1].wait()
                    gd[jn] = pltpu.async_copy(y_hbm.at[src_v.at[jn]],
                                              rows_v.at[q], gsems[q])
                gd[j].wait()
                sd[j] = pltpu.async_copy(rows_v.at[p], acc.at[dst_v.at[j]],
                                         ssems[p], add=True)
            for j in range(max(0, nb - nbuf), nb):
                sd[j].wait()
            return carry

        lax.fori_loop(0, nblk, step_blk, 0)
        plsc.subcore_barrier()
        _sliced_copy(sid, rps, rem, acc, out_hbm.at[cid])

    return pl.kernel(body,
                     out_type=(jax.ShapeDtypeStruct((_NC, n, d),
                                                    jnp.float32),),
                     mesh=mesh, scratch_types=scratch,
                     compiler_params=pltpu.CompilerParams(
                         use_tc_tiling_on_sc=False))(y, src_r, dst_r,
                                                     zeros_d)[0]


def kernel(x, edge_index, W1l, b1l, W1r, g1, be1, W2l, b2l, W2r, g2, be2,
           W3l, b3l, W3r):
    n = x.shape[0]
    e = edge_index.shape[1]
    nch = e // (_NW * _C)
    nch3 = e // (_NW * _C3)
    src_r = edge_index[0].reshape(_NW, nch, _C)
    dst_r = edge_index[1].reshape(_NW, nch, _C)
    src_r3 = edge_index[0].reshape(_NW, nch3, _C3)
    dst_r3 = edge_index[1].reshape(_NW, nch3, _C3)
    z128 = jnp.zeros((n, W1l.shape[0]), jnp.float32)
    z16 = jnp.zeros((n, 16), jnp.float32)
    ones16 = jnp.ones((_C3, 16), jnp.float32)

    cp = _sc_count(dst_r3, z16, ones16)
    y1 = _tc_mm(x, W1l)
    s1p = _sc_agg(y1, src_r, dst_r, z128)
    z1 = _tc_mm(x, W1r)
    y2, h1 = _tc_mid(s1p, cp, z1, b1l, g1, be1, W2l)
    s2p = _sc_agg(y2, src_r, dst_r, z128)
    z2 = _tc_mm(h1, W2r)
    y3, h2 = _tc_mid(s2p, cp, z2, b2l, g2, be2, W3l)
    s3p = _sc_agg(y3, src_r3, dst_r3, z16)
    z3 = _tc_mm(h2, W3r)
    return _tc_post(s3p, cp, z3, b3l)


# speedup vs baseline: 1.0082x; 1.0082x over previous
import jax
import jax.numpy as jnp
from jax import lax
from jax.experimental import pallas as pl
from jax.experimental.pallas import tpu as pltpu
from jax.experimental.pallas import tpu_sc as plsc

_NC = 2
_NS = 16
_NW = _NC * _NS
_C = 125
_C3 = 625
_NB = 16


def _dot_t(a, w):
    return lax.dot_general(a, w, (((1,), (1,)), ((), ())),
                           preferred_element_type=jnp.float32)


def _tc_mm(a, w):
    n = a.shape[0]
    do = w.shape[0]

    def body(a_ref, w_ref, y_ref):
        y_ref[...] = _dot_t(a_ref[...], w_ref[...])

    return pl.pallas_call(
        body,
        out_shape=jax.ShapeDtypeStruct((n, do), jnp.float32),
    )(a, w)


def _tc_mid(sp, cp, z, b, g, be, wl, wa, wb):
    n = sp.shape[1]
    da = sp.shape[2] if wa is None else wa.shape[0]
    db = wb.shape[0]

    def body(sp_ref, cp_ref, z_ref, b_ref, g_ref, be_ref, wl_ref, wa_ref,
             wb_ref, a_ref, b2_ref):
        cnt = cp_ref[0] + cp_ref[1]
        inv = 1.0 / jnp.maximum(cnt[:, 0:1], 1.0)
        mean = (sp_ref[0] + sp_ref[1]) * inv
        u = _dot_t(mean, wl_ref[...]) + b_ref[...][None, :] + z_ref[...]
        m = jnp.mean(u, axis=0, keepdims=True)
        v = jnp.mean((u - m) ** 2, axis=0, keepdims=True)
        h = (u - m) * lax.rsqrt(v + 1e-5) * g_ref[...][None, :] + be_ref[...][None, :]
        h = jnp.maximum(h, 0.0)
        a_ref[...] = h if wa is None else _dot_t(h, wa_ref[...])
        b2_ref[...] = _dot_t(h, wb_ref[...])

    args = (sp, cp, z, b, g, be, wl, wb) if wa is None \
        else (sp, cp, z, b, g, be, wl, wa, wb)

    def body_no_wa(sp_ref, cp_ref, z_ref, b_ref, g_ref, be_ref, wl_ref,
                   wb_ref, a_ref, b2_ref):
        return body(sp_ref, cp_ref, z_ref, b_ref, g_ref, be_ref, wl_ref,
                    None, wb_ref, a_ref, b2_ref)

    return pl.pallas_call(
        body if wa is not None else body_no_wa,
        out_shape=(jax.ShapeDtypeStruct((n, da), jnp.float32),
                   jax.ShapeDtypeStruct((n, db), jnp.float32)),
    )(*args)


def _tc_post(sp, cp, z, b):
    n = sp.shape[1]
    do = sp.shape[2]

    def body(sp_ref, cp_ref, z_ref, b_ref, o_ref):
        cnt = cp_ref[0] + cp_ref[1]
        inv = 1.0 / jnp.maximum(cnt[:, 0:1], 1.0)
        u = (sp_ref[0] + sp_ref[1]) * inv + b_ref[...][None, :] + z_ref[...]
        mx = jnp.max(u, axis=1, keepdims=True)
        lse = mx + jnp.log(jnp.sum(jnp.exp(u - mx), axis=1, keepdims=True))
        o_ref[...] = u - lse

    return pl.pallas_call(
        body,
        out_shape=jax.ShapeDtypeStruct((n, do), jnp.float32),
    )(sp, cp, z, b)


def _sliced_copy(sid, rps, rem, src_ref, dst_ref):
    zb = sid * rps
    pltpu.sync_copy(src_ref.at[pl.ds(zb, rps)], dst_ref.at[pl.ds(zb, rps)])
    if rem:
        @pl.when(sid == 0)
        def _():
            pltpu.sync_copy(src_ref.at[pl.ds(_NS * rps, rem)],
                            dst_ref.at[pl.ds(_NS * rps, rem)])


def _sc_count(dst_r, zeros16, ones16):
    n = zeros16.shape[0]
    nch, c = dst_r.shape[1], dst_r.shape[2]
    nb = min(_NB, nch)
    nblk = nch // nb
    rps = (n // _NS) & ~7
    rem = n - _NS * rps
    mesh = plsc.VectorSubcoreMesh(core_axis_name="c", subcore_axis_name="s",
                                  num_cores=_NC, num_subcores=_NS)
    scratch = [
        pltpu.VMEM((nb, c), jnp.int32),
        pltpu.VMEM((c, 16), jnp.float32),
        pltpu.VMEM_SHARED((n, 16), jnp.float32),
        pltpu.SemaphoreType.DMA,
    ]

    def body(dst_hbm, z16_hbm, ones_hbm, cnt_hbm, dst_v, ones_v, cacc, sem):
        cid = lax.axis_index("c")
        sid = lax.axis_index("s")
        wid = cid * _NS + sid
        _sliced_copy(sid, rps, rem, z16_hbm, cacc)
        pltpu.sync_copy(ones_hbm, ones_v)
        plsc.subcore_barrier()

        def step_blk(bi, carry):
            pltpu.sync_copy(dst_hbm.at[wid, pl.ds(bi * nb, nb)], dst_v)
            descs = [pltpu.async_copy(ones_v, cacc.at[dst_v.at[j]], sem,
                                      add=True)
                     for j in range(nb)]
            for dsc in descs:
                dsc.wait()
            return carry

        lax.fori_loop(0, nblk, step_blk, 0)
        plsc.subcore_barrier()
        _sliced_copy(sid, rps, rem, cacc, cnt_hbm.at[cid])

    return pl.kernel(body,
                     out_type=(jax.ShapeDtypeStruct((_NC, n, 16),
                                                    jnp.float32),),
                     mesh=mesh, scratch_types=scratch,
                     compiler_params=pltpu.CompilerParams(
                         use_tc_tiling_on_sc=False))(dst_r, zeros16, ones16)[0]


def _sc_agg(y, src_r, dst_r, zeros_d):
    n, d = y.shape
    nch, c = src_r.shape[1], src_r.shape[2]
    nb = min(_NB, nch)
    nblk = nch // nb
    rps = (n // _NS) & ~7
    rem = n - _NS * rps

    mesh = plsc.VectorSubcoreMesh(core_axis_name="c", subcore_axis_name="s",
                                  num_cores=_NC, num_subcores=_NS)
    nbuf = 3 if d <= 16 else 2
    scratch = [
        pltpu.VMEM((nb, c), jnp.int32),
        pltpu.VMEM((nb, c), jnp.int32),
        pltpu.VMEM((nbuf, c, d), jnp.float32),
        pltpu.VMEM_SHARED((n, d), jnp.float32),
    ] + [pltpu.SemaphoreType.DMA] * (2 * nbuf)

    def body(y_hbm, src_hbm, dst_hbm, zd_hbm, out_hbm,
             src_v, dst_v, rows_v, acc, *sems):
        cid = lax.axis_index("c")
        sid = lax.axis_index("s")
        wid = cid * _NS + sid
        gsems = sems[:nbuf]
        ssems = sems[nbuf:]
        _sliced_copy(sid, rps, rem, zd_hbm, acc)
        plsc.subcore_barrier()

        def step_blk(bi, carry):
            pltpu.sync_copy(src_hbm.at[wid, pl.ds(bi * nb, nb)], src_v)
            pltpu.sync_copy(dst_hbm.at[wid, pl.ds(bi * nb, nb)], dst_v)
            gd = [None] * nb
            sd = [None] * nb
            for j in range(min(nbuf - 1, nb)):
                gd[j] = pltpu.async_copy(y_hbm.at[src_v.at[j]],
                                         rows_v.at[j % nbuf], gsems[j % nbuf])
            for j in range(nb):
                p = j % nbuf
                jn = j + nbuf - 1
                if jn < nb:
                    q = jn % nbuf
                    if j >= 1:
                        sd[j - 1].wait()
                    gd[jn] = pltpu.async_copy(y_hbm.at[src_v.at[jn]],
                                              rows_v.at[q], gsems[q])
                gd[j].wait()
                sd[j] = pltpu.async_copy(rows_v.at[p], acc.at[dst_v.at[j]],
                                         ssems[p], add=True)
            for j in range(max(0, nb - nbuf), nb):
                sd[j].wait()
            return carry

        lax.fori_loop(0, nblk, step_blk, 0)
        plsc.subcore_barrier()
        _sliced_copy(sid, rps, rem, acc, out_hbm.at[cid])

    return pl.kernel(body,
                     out_type=(jax.ShapeDtypeStruct((_NC, n, d),
                                                    jnp.float32),),
                     mesh=mesh, scratch_types=scratch,
                     compiler_params=pltpu.CompilerParams(
                         use_tc_tiling_on_sc=False))(y, src_r, dst_r,
                                                     zeros_d)[0]


def kernel(x, edge_index, W1l, b1l, W1r, g1, be1, W2l, b2l, W2r, g2, be2,
           W3l, b3l, W3r):
    n = x.shape[0]
    e = edge_index.shape[1]
    nch = e // (_NW * _C)
    nch3 = e // (_NW * _C3)
    src_r = edge_index[0].reshape(_NW, nch, _C)
    dst_r = edge_index[1].reshape(_NW, nch, _C)
    src_r3 = edge_index[0].reshape(_NW, nch3, _C3)
    dst_r3 = edge_index[1].reshape(_NW, nch3, _C3)
    z128 = jnp.zeros((n, W1l.shape[0]), jnp.float32)
    z16 = jnp.zeros((n, 16), jnp.float32)
    ones16 = jnp.ones((_C3, 16), jnp.float32)

    cp = _sc_count(dst_r3, z16, ones16)
    s1p = _sc_agg(x, src_r, dst_r, z128)
    z1 = _tc_mm(x, W1r)
    h1, z2 = _tc_mid(s1p, cp, z1, b1l, g1, be1, W1l, None, W2r)
    s2p = _sc_agg(h1, src_r, dst_r, z128)
    y3, z3 = _tc_mid(s2p, cp, z2, b2l, g2, be2, W2l, W3l, W3r)
    s3p = _sc_agg(y3, src_r3, dst_r3, z16)
    return _tc_post(s3p, cp, z3, b3l)
